# baseline (speedup 1.0000x reference)
import jax
import jax.numpy as jnp
from jax.experimental import pallas as pl
from jax.experimental.pallas import tpu as pltpu


def _body(ms_ref, x_ref, p0_ref, p1_ref, o_ref):
    g = pl.program_id(0)
    m = ms_ref[g, 0]
    rows = jax.lax.broadcasted_iota(jnp.int32, (x_ref.shape[1], x_ref.shape[2]), 0)
    s = x_ref[0] + p0_ref[0] + p1_ref[0]
    o_ref[0] = jnp.where(rows < m, jnp.maximum(s, 0.0), 0.0)


def kernel(atom_features, mol_slice, prev_activations):
    B, A, F = atom_features.shape
    p0 = prev_activations[0]
    p1 = prev_activations[1]
    return pl.pallas_call(
        _body,
        grid=(B,),
        in_specs=[
            pl.BlockSpec(memory_space=pltpu.SMEM),
            pl.BlockSpec((1, A, F), lambda g: (g, 0, 0)),
            pl.BlockSpec((1, A, F), lambda g: (g, 0, 0)),
            pl.BlockSpec((1, A, F), lambda g: (g, 0, 0)),
        ],
        out_specs=pl.BlockSpec((1, A, F), lambda g: (g, 0, 0)),
        out_shape=jax.ShapeDtypeStruct((B, A, F), jnp.float32),
    )(mol_slice, atom_features, p0, p1)


# speedup vs baseline: 1.8578x; 1.8578x over previous
import functools

import jax
import jax.numpy as jnp
from jax import lax
from jax.experimental import pallas as pl
from jax.experimental.pallas import tpu as pltpu
from jax.experimental.pallas import tpu_sc as plsc

B, A, F = 256, 128, 128
R = 16
NCHUNK = A // R
NW = 32
GPW = B // NW
NV = F // 16


def _sc_body(x_hbm, ms_hbm, prev_hbm, out_hbm, ms_v, xb, p0b, p1b, ob, zb, sem):
    wid = lax.axis_index("s") * 2 + lax.axis_index("c")
    g0 = wid * GPW
    pltpu.sync_copy(ms_hbm.at[pl.ds(g0 * 2, 2 * GPW)], ms_v)
    mvec = ms_v[...]

    zvec = jnp.zeros((16,), jnp.float32)
    for j in range(R):
        for k in range(NV):
            zb[j, pl.ds(k * 16, 16)] = zvec

    for i in range(GPW):
        g = g0 + i
        m = mvec[2 * i]
        nvc = (m + R - 1) // R

        def chunk_body(c, _, g=g, m=m):
            r0 = c * R
            cx = pltpu.make_async_copy(x_hbm.at[g, pl.ds(r0, R), :], xb, sem)
            c0 = pltpu.make_async_copy(prev_hbm.at[0, g, pl.ds(r0, R), :], p0b, sem)
            c1 = pltpu.make_async_copy(prev_hbm.at[1, g, pl.ds(r0, R), :], p1b, sem)
            cx.start(); c0.start(); c1.start()
            cx.wait(); c0.wait(); c1.wait()

            def row_body(j, _):
                valid = (r0 + j) < m
                for k in range(NV):
                    sl = pl.ds(k * 16, 16)
                    v = xb[j, sl] + p0b[j, sl] + p1b[j, sl]
                    ob[j, sl] = jnp.where(valid, jnp.maximum(v, 0.0), 0.0)
                return 0

            lax.fori_loop(0, R, row_body, 0, unroll=False)
            pltpu.sync_copy(ob, out_hbm.at[g, pl.ds(r0, R), :])
            return 0

        lax.fori_loop(0, nvc, chunk_body, 0, unroll=False)

        def zero_body(c, _, g=g):
            pltpu.sync_copy(zb, out_hbm.at[g, pl.ds(c * R, R), :])
            return 0

        lax.fori_loop(nvc, NCHUNK, zero_body, 0, unroll=False)


def kernel(atom_features, mol_slice, prev_activations):
    mesh = plsc.VectorSubcoreMesh(core_axis_name="c", subcore_axis_name="s")
    run = functools.partial(
        pl.kernel,
        mesh=mesh,
        out_type=jax.ShapeDtypeStruct((B, A, F), jnp.float32),
        scratch_types=[
            pltpu.VMEM((2 * GPW,), jnp.int32),
            pltpu.VMEM((R, F), jnp.float32),
            pltpu.VMEM((R, F), jnp.float32),
            pltpu.VMEM((R, F), jnp.float32),
            pltpu.VMEM((R, F), jnp.float32),
            pltpu.VMEM((R, F), jnp.float32),
            pltpu.SemaphoreType.DMA,
        ],
    )(_sc_body)
    return run(atom_features, mol_slice.reshape(-1), prev_activations)


# speedup vs baseline: 1.8853x; 1.0148x over previous
import functools

import jax
import jax.numpy as jnp
from jax import lax
from jax.experimental import pallas as pl
from jax.experimental.pallas import tpu as pltpu
from jax.experimental.pallas import tpu_sc as plsc

B, A, F = 256, 128, 128
R = 16
NCHUNK = A // R
NW = 32
GPW = B // NW
NV = F // 16
ZR = 64


def _sc_body(x_hbm, ms_hbm, prev_hbm, out_hbm,
             ms_v, xb, p0b, p1b, ob, zb, sem_in, sem_out, sem_z):
    wid = lax.axis_index("s") * 2 + lax.axis_index("c")
    g0 = wid * GPW
    pltpu.sync_copy(ms_hbm.at[pl.ds(g0 * 2, 2 * GPW)], ms_v)
    mvec = ms_v[...]

    zvec = jnp.zeros((16,), jnp.float32)
    for j in range(ZR):
        for k in range(NV):
            zb[j, pl.ds(k * 16, 16)] = zvec

    for i in range(GPW):
        g = g0 + i
        m = mvec[2 * i]
        nvc = (m + R - 1) // R
        t = A - nvc * R
        base = pl.multiple_of(nvc * R, R)

        def z64(g=g, base=base):
            pltpu.make_async_copy(
                zb.at[pl.ds(0, 64), :], out_hbm.at[g, pl.ds(base, 64), :],
                sem_z).start()
        pl.when((t & 64) != 0)(z64)
        off32 = pl.multiple_of(base + (t & 64), R)

        def z32(g=g, off32=off32):
            pltpu.make_async_copy(
                zb.at[pl.ds(0, 32), :], out_hbm.at[g, pl.ds(off32, 32), :],
                sem_z).start()
        pl.when((t & 32) != 0)(z32)
        off16 = pl.multiple_of(off32 + (t & 32), R)

        def z16(g=g, off16=off16):
            pltpu.make_async_copy(
                zb.at[pl.ds(0, 16), :], out_hbm.at[g, pl.ds(off16, 16), :],
                sem_z).start()
        pl.when((t & 16) != 0)(z16)

        def in_copies(c, b, g=g):
            r0 = pl.multiple_of(c * R, R)
            return (
                pltpu.make_async_copy(x_hbm.at[g, pl.ds(r0, R), :],
                                      xb.at[b], sem_in.at[b]),
                pltpu.make_async_copy(prev_hbm.at[0, g, pl.ds(r0, R), :],
                                      p0b.at[b], sem_in.at[b]),
                pltpu.make_async_copy(prev_hbm.at[1, g, pl.ds(r0, R), :],
                                      p1b.at[b], sem_in.at[b]),
            )

        def out_copy(c, b, g=g):
            return pltpu.make_async_copy(
                ob.at[b], out_hbm.at[g, pl.ds(pl.multiple_of(c * R, R), R), :],
                sem_out.at[b])

        for cp in in_copies(0, 0):
            cp.start()

        def chunk_body(c, _, g=g, m=m, nvc=nvc):
            b = lax.rem(c, 2)

            def prefetch(c=c, b=b):
                for cp in in_copies(c + 1, 1 - b):
                    cp.start()
            pl.when(c + 1 < nvc)(prefetch)

            for cp in in_copies(c, b):
                cp.wait()

            def drain_prev_out(c=c, b=b):
                out_copy(c - 2, b).wait()
            pl.when(c >= 2)(drain_prev_out)

            r0 = c * R

            def row_body(j, _):
                valid = (r0 + j) < m
                for k in range(NV):
                    sl = pl.ds(k * 16, 16)
                    v = xb[b, j, sl] + p0b[b, j, sl] + p1b[b, j, sl]
                    ob[b, j, sl] = jnp.where(valid, jnp.maximum(v, 0.0), 0.0)
                return 0

            lax.fori_loop(0, R, row_body, 0)
            out_copy(c, b).start()
            return 0

        lax.fori_loop(0, nvc, chunk_body, 0)

        def drain_m2(nvc=nvc, g=g):
            out_copy(nvc - 2, lax.rem(nvc - 2, 2)).wait()
        pl.when(nvc >= 2)(drain_m2)
        out_copy(nvc - 1, lax.rem(nvc - 1, 2)).wait()

        def zw64(g=g, base=base):
            pltpu.make_async_copy(
                zb.at[pl.ds(0, 64), :], out_hbm.at[g, pl.ds(base, 64), :],
                sem_z).wait()
        pl.when((t & 64) != 0)(zw64)

        def zw32(g=g, off32=off32):
            pltpu.make_async_copy(
                zb.at[pl.ds(0, 32), :], out_hbm.at[g, pl.ds(off32, 32), :],
                sem_z).wait()
        pl.when((t & 32) != 0)(zw32)

        def zw16(g=g, off16=off16):
            pltpu.make_async_copy(
                zb.at[pl.ds(0, 16), :], out_hbm.at[g, pl.ds(off16, 16), :],
                sem_z).wait()
        pl.when((t & 16) != 0)(zw16)


def kernel(atom_features, mol_slice, prev_activations):
    mesh = plsc.VectorSubcoreMesh(core_axis_name="c", subcore_axis_name="s")
    run = functools.partial(
        pl.kernel,
        mesh=mesh,
        out_type=jax.ShapeDtypeStruct((B, A, F), jnp.float32),
        scratch_types=[
            pltpu.VMEM((2 * GPW,), jnp.int32),
            pltpu.VMEM((2, R, F), jnp.float32),
            pltpu.VMEM((2, R, F), jnp.float32),
            pltpu.VMEM((2, R, F), jnp.float32),
            pltpu.VMEM((2, R, F), jnp.float32),
            pltpu.VMEM((ZR, F), jnp.float32),
            pltpu.SemaphoreType.DMA((2,)),
            pltpu.SemaphoreType.DMA((2,)),
            pltpu.SemaphoreType.DMA,
        ],
    )(_sc_body)
    return run(atom_features, mol_slice.reshape(-1), prev_activations)


# speedup vs baseline: 2.5932x; 1.3755x over previous
import functools

import jax
import jax.numpy as jnp
from jax import lax
from jax.experimental import pallas as pl
from jax.experimental.pallas import tpu as pltpu
from jax.experimental.pallas import tpu_sc as plsc

B, A, F = 256, 128, 128
R = 16
NCHUNK = A // R
NW = 32
GPW = B // NW
NV = F // 16
ZR = 64


def _sc_body(x_hbm, ms_hbm, prev_hbm, out_hbm,
             ms_v, xb, p0b, p1b, ob, zb, sem_in, sem_out, sem_z):
    wid = lax.axis_index("s") * 2 + lax.axis_index("c")
    g0 = wid * GPW
    pltpu.sync_copy(ms_hbm.at[pl.ds(g0 * 2, 2 * GPW)], ms_v)
    mvec = ms_v[...]

    zvec = jnp.zeros((16,), jnp.float32)
    for j in range(ZR):
        for k in range(NV):
            zb[j, pl.ds(k * 16, 16)] = zvec

    for i in range(GPW):
        g = g0 + i
        m = mvec[2 * i]
        nvc = (m + R - 1) // R
        t = A - nvc * R
        base = pl.multiple_of(nvc * R, R)

        def z64(g=g, base=base):
            pltpu.make_async_copy(
                zb.at[pl.ds(0, 64), :], out_hbm.at[g, pl.ds(base, 64), :],
                sem_z).start()
        pl.when((t & 64) != 0)(z64)
        off32 = pl.multiple_of(base + (t & 64), R)

        def z32(g=g, off32=off32):
            pltpu.make_async_copy(
                zb.at[pl.ds(0, 32), :], out_hbm.at[g, pl.ds(off32, 32), :],
                sem_z).start()
        pl.when((t & 32) != 0)(z32)
        off16 = pl.multiple_of(off32 + (t & 32), R)

        def z16(g=g, off16=off16):
            pltpu.make_async_copy(
                zb.at[pl.ds(0, 16), :], out_hbm.at[g, pl.ds(off16, 16), :],
                sem_z).start()
        pl.when((t & 16) != 0)(z16)

        def in_copies(c, b, g=g):
            r0 = pl.multiple_of(c * R, R)
            return (
                pltpu.make_async_copy(x_hbm.at[g, pl.ds(r0, R), :],
                                      xb.at[b], sem_in.at[b]),
                pltpu.make_async_copy(prev_hbm.at[0, g, pl.ds(r0, R), :],
                                      p0b.at[b], sem_in.at[b]),
                pltpu.make_async_copy(prev_hbm.at[1, g, pl.ds(r0, R), :],
                                      p1b.at[b], sem_in.at[b]),
            )

        def out_copy(c, b, g=g):
            return pltpu.make_async_copy(
                ob.at[b], out_hbm.at[g, pl.ds(pl.multiple_of(c * R, R), R), :],
                sem_out.at[b])

        for cp in in_copies(0, 0):
            cp.start()

        def chunk_body(c, _, g=g, m=m, nvc=nvc):
            b = lax.rem(c, 2)

            def prefetch(c=c, b=b):
                for cp in in_copies(c + 1, 1 - b):
                    cp.start()
            pl.when(c + 1 < nvc)(prefetch)

            for cp in in_copies(c, b):
                cp.wait()

            def drain_prev_out(c=c, b=b):
                out_copy(c - 2, b).wait()
            pl.when(c >= 2)(drain_prev_out)

            r0 = c * R

            @plsc.parallel_loop(0, R, step=1, unroll=4)
            def row_body(j):
                valid = (r0 + j) < m
                for k in range(NV):
                    sl = pl.ds(k * 16, 16)
                    v = xb[b, j, sl] + p0b[b, j, sl] + p1b[b, j, sl]
                    ob[b, j, sl] = jnp.where(valid, jnp.maximum(v, 0.0), 0.0)
            out_copy(c, b).start()
            return 0

        lax.fori_loop(0, nvc, chunk_body, 0)

        def drain_m2(nvc=nvc, g=g):
            out_copy(nvc - 2, lax.rem(nvc - 2, 2)).wait()
        pl.when(nvc >= 2)(drain_m2)
        out_copy(nvc - 1, lax.rem(nvc - 1, 2)).wait()

        def zw64(g=g, base=base):
            pltpu.make_async_copy(
                zb.at[pl.ds(0, 64), :], out_hbm.at[g, pl.ds(base, 64), :],
                sem_z).wait()
        pl.when((t & 64) != 0)(zw64)

        def zw32(g=g, off32=off32):
            pltpu.make_async_copy(
                zb.at[pl.ds(0, 32), :], out_hbm.at[g, pl.ds(off32, 32), :],
                sem_z).wait()
        pl.when((t & 32) != 0)(zw32)

        def zw16(g=g, off16=off16):
            pltpu.make_async_copy(
                zb.at[pl.ds(0, 16), :], out_hbm.at[g, pl.ds(off16, 16), :],
                sem_z).wait()
        pl.when((t & 16) != 0)(zw16)


def kernel(atom_features, mol_slice, prev_activations):
    mesh = plsc.VectorSubcoreMesh(core_axis_name="c", subcore_axis_name="s")
    run = functools.partial(
        pl.kernel,
        mesh=mesh,
        out_type=jax.ShapeDtypeStruct((B, A, F), jnp.float32),
        scratch_types=[
            pltpu.VMEM((2 * GPW,), jnp.int32),
            pltpu.VMEM((2, R, F), jnp.float32),
            pltpu.VMEM((2, R, F), jnp.float32),
            pltpu.VMEM((2, R, F), jnp.float32),
            pltpu.VMEM((2, R, F), jnp.float32),
            pltpu.VMEM((ZR, F), jnp.float32),
            pltpu.SemaphoreType.DMA((2,)),
            pltpu.SemaphoreType.DMA((2,)),
            pltpu.SemaphoreType.DMA,
        ],
    )(_sc_body)
    return run(atom_features, mol_slice.reshape(-1), prev_activations)


# speedup vs baseline: 2.8296x; 1.0912x over previous
import functools

import jax
import jax.numpy as jnp
from jax import lax
from jax.experimental import pallas as pl
from jax.experimental.pallas import tpu as pltpu
from jax.experimental.pallas import tpu_sc as plsc

B, A, F = 256, 128, 128
R = 16
NCHUNK = A // R
NW = 32
GPW = B // NW
NV = F // 16
ZR = 64


def _sc_body(x_hbm, ms_hbm, prev_hbm, out_hbm,
             ms_v, xb, p0b, p1b, ob, zb, ms_s, sem_in, sem_out, sem_z):
    wid = lax.axis_index("s") * 2 + lax.axis_index("c")
    g0 = pl.multiple_of(wid * GPW, GPW)
    pltpu.sync_copy(ms_hbm.at[pl.ds(g0 * 2, 2 * GPW)], ms_v)
    mvec = ms_v[...]
    for i in range(GPW):
        ms_s[i] = mvec[2 * i]

    zvec = jnp.zeros((16,), jnp.float32)
    for j in range(ZR):
        for k in range(NV):
            zb[j, pl.ds(k * 16, 16)] = zvec

    def graph_body(i, _):
        g = g0 + i
        m = ms_s[i]
        nvc = (m + R - 1) // R
        t = A - nvc * R
        base = pl.multiple_of(nvc * R, R)

        def z64():
            pltpu.make_async_copy(
                zb.at[pl.ds(0, 64), :], out_hbm.at[g, pl.ds(base, 64), :],
                sem_z).start()
        pl.when((t & 64) != 0)(z64)
        off32 = pl.multiple_of(base + (t & 64), R)

        def z32():
            pltpu.make_async_copy(
                zb.at[pl.ds(0, 32), :], out_hbm.at[g, pl.ds(off32, 32), :],
                sem_z).start()
        pl.when((t & 32) != 0)(z32)
        off16 = pl.multiple_of(off32 + (t & 32), R)

        def z16():
            pltpu.make_async_copy(
                zb.at[pl.ds(0, 16), :], out_hbm.at[g, pl.ds(off16, 16), :],
                sem_z).start()
        pl.when((t & 16) != 0)(z16)

        def in_copies(c, b):
            r0 = pl.multiple_of(c * R, R)
            return (
                pltpu.make_async_copy(x_hbm.at[g, pl.ds(r0, R), :],
                                      xb.at[b], sem_in.at[b]),
                pltpu.make_async_copy(prev_hbm.at[0, g, pl.ds(r0, R), :],
                                      p0b.at[b], sem_in.at[b]),
                pltpu.make_async_copy(prev_hbm.at[1, g, pl.ds(r0, R), :],
                                      p1b.at[b], sem_in.at[b]),
            )

        def out_copy(c, b):
            return pltpu.make_async_copy(
                ob.at[b], out_hbm.at[g, pl.ds(pl.multiple_of(c * R, R), R), :],
                sem_out.at[b])

        for cp in in_copies(0, 0):
            cp.start()

        def chunk_body(c, _):
            b = lax.rem(c, 2)

            def prefetch():
                for cp in in_copies(c + 1, 1 - b):
                    cp.start()
            pl.when(c + 1 < nvc)(prefetch)

            for cp in in_copies(c, b):
                cp.wait()

            def drain_prev_out():
                out_copy(c - 2, b).wait()
            pl.when(c >= 2)(drain_prev_out)

            r0 = c * R

            @plsc.parallel_loop(0, R, step=1, unroll=4)
            def row_body(j):
                valid = (r0 + j) < m
                for k in range(NV):
                    sl = pl.ds(k * 16, 16)
                    v = xb[b, j, sl] + p0b[b, j, sl] + p1b[b, j, sl]
                    ob[b, j, sl] = jnp.where(valid, jnp.maximum(v, 0.0), 0.0)

            out_copy(c, b).start()
            return 0

        lax.fori_loop(0, nvc, chunk_body, 0)

        def drain_m2():
            out_copy(nvc - 2, lax.rem(nvc - 2, 2)).wait()
        pl.when(nvc >= 2)(drain_m2)
        out_copy(nvc - 1, lax.rem(nvc - 1, 2)).wait()

        def zw64():
            pltpu.make_async_copy(
                zb.at[pl.ds(0, 64), :], out_hbm.at[g, pl.ds(base, 64), :],
                sem_z).wait()
        pl.when((t & 64) != 0)(zw64)

        def zw32():
            pltpu.make_async_copy(
                zb.at[pl.ds(0, 32), :], out_hbm.at[g, pl.ds(off32, 32), :],
                sem_z).wait()
        pl.when((t & 32) != 0)(zw32)

        def zw16():
            pltpu.make_async_copy(
                zb.at[pl.ds(0, 16), :], out_hbm.at[g, pl.ds(off16, 16), :],
                sem_z).wait()
        pl.when((t & 16) != 0)(zw16)
        return 0

    lax.fori_loop(0, GPW, graph_body, 0)


def kernel(atom_features, mol_slice, prev_activations):
    mesh = plsc.VectorSubcoreMesh(core_axis_name="c", subcore_axis_name="s")
    run = functools.partial(
        pl.kernel,
        mesh=mesh,
        out_type=jax.ShapeDtypeStruct((B, A, F), jnp.float32),
        scratch_types=[
            pltpu.VMEM((2 * GPW,), jnp.int32),
            pltpu.VMEM((2, R, F), jnp.float32),
            pltpu.VMEM((2, R, F), jnp.float32),
            pltpu.VMEM((2, R, F), jnp.float32),
            pltpu.VMEM((2, R, F), jnp.float32),
            pltpu.VMEM((ZR, F), jnp.float32),
            pltpu.SMEM((GPW,), jnp.int32),
            pltpu.SemaphoreType.DMA((2,)),
            pltpu.SemaphoreType.DMA((2,)),
            pltpu.SemaphoreType.DMA,
        ],
    )(_sc_body)
    return run(atom_features, mol_slice.reshape(-1), prev_activations)
